# baseline (speedup 1.0000x reference)
import functools

import jax
import jax.numpy as jnp
from jax import lax
from jax.experimental import pallas as pl
from jax.experimental.pallas import tpu as pltpu
from jax.experimental.pallas import tpu_sc as plsc

NUM_EMB = 1_000_000
DIM = 64
B = 4096 * 200

NC = 2
NS = 16
NW = NC * NS
B_PER_W = B // NW
CHUNK = 512
N_CHUNKS = B_PER_W // CHUNK

_mesh = plsc.VectorSubcoreMesh(core_axis_name="c", subcore_axis_name="s")


@functools.partial(
    pl.kernel,
    mesh=_mesh,
    out_type=jax.ShapeDtypeStruct((B, DIM), jnp.float32),
    scratch_types=[
        pltpu.VMEM((CHUNK,), jnp.int32),
        pltpu.VMEM((CHUNK, DIM), jnp.float32),
        pltpu.SemaphoreType.DMA,
    ],
    compiler_params=pltpu.CompilerParams(use_tc_tiling_on_sc=False),
)
def _gather_kernel(idx_hbm, table_hbm, out_hbm, idx_v, rows_v, sem):
    wid = lax.axis_index("s") * NC + lax.axis_index("c")
    base = wid * B_PER_W

    def body(i, carry):
        off = base + i * CHUNK
        pltpu.sync_copy(idx_hbm.at[pl.ds(off, CHUNK)], idx_v)
        pltpu.async_copy(table_hbm.at[idx_v], rows_v, sem).wait()
        pltpu.sync_copy(rows_v, out_hbm.at[pl.ds(off, CHUNK)])
        return carry

    lax.fori_loop(0, N_CHUNKS, body, 0)


def kernel(token_ids, weight):
    idx = token_ids.reshape(-1).astype(jnp.int32)
    out = _gather_kernel(idx, weight)
    return out.reshape(*token_ids.shape, DIM)


# speedup vs baseline: 1.0432x; 1.0432x over previous
import functools

import jax
import jax.numpy as jnp
from jax import lax
from jax.experimental import pallas as pl
from jax.experimental.pallas import tpu as pltpu
from jax.experimental.pallas import tpu_sc as plsc

NUM_EMB = 1_000_000
DIM = 64
B = 4096 * 200

NC = 2
NS = 16
NW = NC * NS
B_PER_W = B // NW
CHUNK = 512
N_CHUNKS = B_PER_W // CHUNK

_mesh = plsc.VectorSubcoreMesh(core_axis_name="c", subcore_axis_name="s")


@functools.partial(
    pl.kernel,
    mesh=_mesh,
    out_type=jax.ShapeDtypeStruct((B, DIM), jnp.float32),
    scratch_types=[
        pltpu.VMEM((B_PER_W,), jnp.int32),
        pltpu.VMEM((CHUNK, DIM), jnp.float32),
        pltpu.VMEM((CHUNK, DIM), jnp.float32),
        pltpu.SemaphoreType.DMA,
        pltpu.SemaphoreType.DMA,
        pltpu.SemaphoreType.DMA,
        pltpu.SemaphoreType.DMA,
    ],
    compiler_params=pltpu.CompilerParams(use_tc_tiling_on_sc=False),
)
def _gather_kernel(idx_hbm, table_hbm, out_hbm, idx_v, rows0, rows1,
                   g0, g1, o0, o1):
    wid = lax.axis_index("s") * NC + lax.axis_index("c")
    base = wid * B_PER_W
    rows = (rows0, rows1)
    gsem = (g0, g1)
    osem = (o0, o1)

    def gather(i, b):
        pltpu.async_copy(
            table_hbm.at[idx_v.at[pl.ds(i * CHUNK, CHUNK)]], rows[b], gsem[b])

    def out(i, b):
        pltpu.async_copy(
            rows[b], out_hbm.at[pl.ds(base + i * CHUNK, CHUNK)], osem[b])

    def wait(sems, b, src, dst):
        pltpu.make_async_copy(src, dst, sems[b]).wait()

    pltpu.sync_copy(idx_hbm.at[pl.ds(base, B_PER_W)], idx_v)
    gather(0, 0)
    gather(1, 1)
    wait(gsem, 0, table_hbm.at[idx_v.at[pl.ds(0, CHUNK)]], rows[0])
    out(0, 0)

    def body(j, carry):
        i = 1 + 2 * j
        wait(osem, 0, rows[0], out_hbm.at[pl.ds(base, CHUNK)])
        gather(i + 1, 0)
        wait(gsem, 1, table_hbm.at[idx_v.at[pl.ds(0, CHUNK)]], rows[1])
        out(i, 1)
        wait(osem, 1, rows[1], out_hbm.at[pl.ds(base, CHUNK)])
        gather(i + 2, 1)
        wait(gsem, 0, table_hbm.at[idx_v.at[pl.ds(0, CHUNK)]], rows[0])
        out(i + 1, 0)
        return carry

    lax.fori_loop(0, (N_CHUNKS - 2) // 2, body, 0)

    wait(gsem, 1, table_hbm.at[idx_v.at[pl.ds(0, CHUNK)]], rows[1])
    out(N_CHUNKS - 1, 1)
    wait(osem, 0, rows[0], out_hbm.at[pl.ds(base, CHUNK)])
    wait(osem, 1, rows[1], out_hbm.at[pl.ds(base, CHUNK)])


def kernel(token_ids, weight):
    idx = token_ids.reshape(-1).astype(jnp.int32)
    out = _gather_kernel(idx, weight)
    return out.reshape(*token_ids.shape, DIM)


# speedup vs baseline: 1.3907x; 1.3330x over previous
import functools

import jax
import jax.numpy as jnp
from jax import lax
from jax.experimental import pallas as pl
from jax.experimental.pallas import tpu as pltpu
from jax.experimental.pallas import tpu_sc as plsc

NUM_EMB = 1_000_000
DIM = 64
B = 4096 * 200

NC = 2
NS = 16
NW = NC * NS
B_PER_W = B // NW
CHUNK = 512
N_CHUNKS = B_PER_W // CHUNK

_mesh = plsc.VectorSubcoreMesh(core_axis_name="c", subcore_axis_name="s")


@functools.partial(
    pl.kernel,
    mesh=_mesh,
    out_type=jax.ShapeDtypeStruct((B, 2 * DIM), jnp.float32),
    scratch_types=[
        pltpu.VMEM((B_PER_W,), jnp.int32),
        pltpu.VMEM((CHUNK, DIM), jnp.float32),
        pltpu.VMEM((CHUNK, DIM), jnp.float32),
        pltpu.SemaphoreType.DMA,
        pltpu.SemaphoreType.DMA,
        pltpu.SemaphoreType.DMA,
        pltpu.SemaphoreType.DMA,
    ],
    compiler_params=pltpu.CompilerParams(use_tc_tiling_on_sc=False),
)
def _gather_kernel(idx_hbm, table_hbm, out_hbm, idx_v, rows0, rows1,
                   g0, g1, o0, o1):
    wid = lax.axis_index("s") * NC + lax.axis_index("c")
    base = wid * B_PER_W
    rows = (rows0, rows1)
    gsem = (g0, g1)
    osem = (o0, o1)

    def gather(i, b):
        pltpu.async_copy(
            table_hbm.at[idx_v.at[pl.ds(i * CHUNK, CHUNK)]], rows[b], gsem[b])

    def out(i, b):
        pltpu.async_copy(
            rows[b],
            out_hbm.at[pl.ds(base + i * CHUNK, CHUNK), pl.ds(0, DIM)],
            osem[b])

    def wait(sems, b, src, dst):
        pltpu.make_async_copy(src, dst, sems[b]).wait()

    pltpu.sync_copy(idx_hbm.at[pl.ds(base, B_PER_W)], idx_v)
    gather(0, 0)
    gather(1, 1)
    wait(gsem, 0, table_hbm.at[idx_v.at[pl.ds(0, CHUNK)]], rows[0])
    out(0, 0)

    def body(j, carry):
        i = 1 + 2 * j
        wait(osem, 0, rows[0], out_hbm.at[pl.ds(base, CHUNK), pl.ds(0, DIM)])
        gather(i + 1, 0)
        wait(gsem, 1, table_hbm.at[idx_v.at[pl.ds(0, CHUNK)]], rows[1])
        out(i, 1)
        wait(osem, 1, rows[1], out_hbm.at[pl.ds(base, CHUNK), pl.ds(0, DIM)])
        gather(i + 2, 1)
        wait(gsem, 0, table_hbm.at[idx_v.at[pl.ds(0, CHUNK)]], rows[0])
        out(i + 1, 0)
        return carry

    lax.fori_loop(0, (N_CHUNKS - 2) // 2, body, 0)

    wait(gsem, 1, table_hbm.at[idx_v.at[pl.ds(0, CHUNK)]], rows[1])
    out(N_CHUNKS - 1, 1)
    wait(osem, 0, rows[0], out_hbm.at[pl.ds(base, CHUNK), pl.ds(0, DIM)])
    wait(osem, 1, rows[1], out_hbm.at[pl.ds(base, CHUNK), pl.ds(0, DIM)])


def kernel(token_ids, weight):
    idx = token_ids.reshape(-1).astype(jnp.int32)
    out = _gather_kernel(idx, weight)
    return out[:, :DIM].reshape(*token_ids.shape, DIM)
